# baseline (speedup 1.0000x reference)
import functools

import jax
import jax.numpy as jnp
from jax import lax
from jax.experimental import pallas as pl
from jax.experimental.pallas import tpu as pltpu
from jax.experimental.pallas import tpu_sc as plsc

SMIN = 0.01
SMAX = 50.0
NUM_LAYERS = 2
MB = 64
MB_ONEHOT = 64

SC_NC = 2
SC_NS = 16
SC_NW = SC_NC * SC_NS


def _sc_gather(table, idx, n_rows, D):
    b_per_w = n_rows // SC_NW
    mesh = plsc.VectorSubcoreMesh(core_axis_name="c", subcore_axis_name="s")

    @functools.partial(
        pl.kernel, mesh=mesh,
        out_type=jax.ShapeDtypeStruct((n_rows, D), jnp.float32),
        scratch_types=[
            pltpu.VMEM((b_per_w,), jnp.int32),
            pltpu.VMEM((b_per_w, D), jnp.float32),
            pltpu.SemaphoreType.DMA,
        ],
    )
    def k(table_hbm, idx_hbm, out_hbm, idx_v, rows_v, sem):
        wid = lax.axis_index("s") * SC_NC + lax.axis_index("c")
        base = wid * b_per_w
        pltpu.sync_copy(idx_hbm.at[pl.ds(base, b_per_w)], idx_v)
        pltpu.async_copy(table_hbm.at[idx_v], rows_v, sem).wait()
        pltpu.sync_copy(rows_v, out_hbm.at[pl.ds(base, b_per_w)])

    return k(table, idx)


def _body(t_nodes, h, pos, noise, A_ref, B_ref, C_ref, Wc_ref, Wn_ref, bn_ref,
          out_ref, *, mb, n, D, n_total):
    NB = mb * n
    std = SMIN * (SMAX / SMIN) ** t_nodes

    noise3 = noise.reshape(mb, n, 3)
    noise_c = (noise3 - jnp.mean(noise3, axis=1, keepdims=True)).reshape(NB, 3)

    x = pos + noise_c * std

    score = jnp.zeros((NB, 3), dtype=jnp.float32)
    for l in range(NUM_LAYERS):
        a = h * A_ref[l][None, :]
        b = h * B_ref[l][None, :]
        Cl = C_ref[l][None, :]
        Wcl = Wc_ref[l].reshape(D, 1)
        a3 = a.reshape(mb, n, D)
        x3 = x.reshape(mb, n, 3)
        agg_m = jnp.zeros((NB, D), dtype=jnp.float32)
        agg_x = jnp.zeros((NB, 3), dtype=jnp.float32)
        for i in range(n):
            a_i = jnp.broadcast_to(a3[:, i:i + 1, :], (mb, n, D))
            x_i = jnp.broadcast_to(x3[:, i:i + 1, :], (mb, n, 3))
            rel = (x_i - x3).reshape(NB, 3)
            d2 = jnp.sum(rel * rel, axis=1, keepdims=True)
            m = jax.nn.silu(a_i.reshape(NB, D) + b + d2 * Cl)
            agg_m = agg_m + m
            coef = jax.lax.dot_general(m, Wcl, (((1,), (0,)), ((), ())),
                                       preferred_element_type=jnp.float32)
            agg_x = agg_x + rel * coef
        agg_x = agg_x / n
        h = h + jax.nn.silu(
            jax.lax.dot_general(agg_m, Wn_ref[l], (((1,), (0,)), ((), ())),
                                preferred_element_type=jnp.float32)
            + bn_ref[l][None, :])
        x = x + agg_x
        score = score + agg_x

    score = score / std
    score3 = score.reshape(mb, n, 3)
    score = (score3 - jnp.mean(score3, axis=1, keepdims=True)).reshape(NB, 3)
    r = score * std + noise_c
    out_ref[...] = (jnp.sum(r * r, axis=(0, 1), keepdims=True)
                    / n_total).reshape(1, 1, 1)


def _step_h0(t_ref, h0_ref, pos_ref, noise_ref, Wt_ref, A_ref, B_ref,
             C_ref, Wc_ref, Wn_ref, bn_ref, out_ref, *, mb, n, D, n_total):
    t_nodes = t_ref[...]
    h = h0_ref[...] + t_nodes * Wt_ref[0][None, :]
    _body(t_nodes, h, pos_ref[...], noise_ref[...], A_ref, B_ref, C_ref,
          Wc_ref, Wn_ref, bn_ref, out_ref, mb=mb, n=n, D=D, n_total=n_total)


def _step_onehot(t_ref, an_ref, pos_ref, noise_ref, emb_ref, Wt_ref, A_ref,
                 B_ref, C_ref, Wc_ref, Wn_ref, bn_ref, out_ref, *, mb, n, D,
                 n_types, n_total):
    NB = mb * n
    t_nodes = t_ref[...]
    oh = (jax.lax.broadcasted_iota(jnp.int32, (NB, n_types), 1)
          == an_ref[...]).astype(jnp.float32)
    h = (jax.lax.dot_general(oh, emb_ref[...], (((1,), (0,)), ((), ())),
                             preferred_element_type=jnp.float32)
         + t_nodes * Wt_ref[0][None, :])
    _body(t_nodes, h, pos_ref[...], noise_ref[...], A_ref, B_ref, C_ref,
          Wc_ref, Wn_ref, bn_ref, out_ref, mb=mb, n=n, D=D, n_total=n_total)


def kernel(pos, atomic_numbers, mask, atom_emb, W_t, A, Bv, C, Wc, Wn, bn):
    B = mask.shape[0]
    N = pos.shape[0]
    n = N // B
    D = atom_emb.shape[1]
    n_types = atom_emb.shape[0]

    kt = jax.random.fold_in(jax.random.key(0), 1)
    kn = jax.random.fold_in(jax.random.key(0), 2)
    t = jax.random.uniform(kt, (B,), minval=1e-3, maxval=1.0, dtype=jnp.float32)
    noise = jax.random.normal(kn, (N, 3), dtype=jnp.float32)

    t_nodes = jnp.repeat(t, n).reshape(N, 1)
    an2 = atomic_numbers.reshape(N, 1)
    Wt2 = W_t.reshape(1, D)

    BA = MB_ONEHOT
    NA = BA * n
    h0_tail = _sc_gather(atom_emb, atomic_numbers[NA:], N - NA, D)

    wspecs = [
        pl.BlockSpec((1, D), lambda g: (0, 0)),
        pl.BlockSpec((NUM_LAYERS, D), lambda g: (0, 0)),
        pl.BlockSpec((NUM_LAYERS, D), lambda g: (0, 0)),
        pl.BlockSpec((NUM_LAYERS, D), lambda g: (0, 0)),
        pl.BlockSpec((NUM_LAYERS, D), lambda g: (0, 0)),
        pl.BlockSpec((NUM_LAYERS, D, D), lambda g: (0, 0, 0)),
        pl.BlockSpec((NUM_LAYERS, D), lambda g: (0, 0)),
    ]

    grid_a = BA // MB
    NBA = MB * n
    out_a = pl.pallas_call(
        functools.partial(_step_onehot, mb=MB, n=n, D=D, n_types=n_types,
                          n_total=N),
        grid=(grid_a,),
        in_specs=[
            pl.BlockSpec((NBA, 1), lambda g: (g, 0)),
            pl.BlockSpec((NBA, 1), lambda g: (g, 0)),
            pl.BlockSpec((NBA, 3), lambda g: (g, 0)),
            pl.BlockSpec((NBA, 3), lambda g: (g, 0)),
            pl.BlockSpec((n_types, D), lambda g: (0, 0)),
        ] + wspecs,
        out_specs=pl.BlockSpec((1, 1, 1), lambda g: (g, 0, 0)),
        out_shape=jax.ShapeDtypeStruct((grid_a, 1, 1), jnp.float32),
    )(t_nodes[:NA], an2[:NA], pos[:NA], noise[:NA], atom_emb,
      Wt2, A, Bv, C, Wc, Wn, bn)

    grid_b = (B - BA) // MB
    out_b = pl.pallas_call(
        functools.partial(_step_h0, mb=MB, n=n, D=D, n_total=N),
        grid=(grid_b,),
        in_specs=[
            pl.BlockSpec((NBA, 1), lambda g: (g, 0)),
            pl.BlockSpec((NBA, D), lambda g: (g, 0)),
            pl.BlockSpec((NBA, 3), lambda g: (g, 0)),
            pl.BlockSpec((NBA, 3), lambda g: (g, 0)),
        ] + wspecs,
        out_specs=pl.BlockSpec((1, 1, 1), lambda g: (g, 0, 0)),
        out_shape=jax.ShapeDtypeStruct((grid_b, 1, 1), jnp.float32),
    )(t_nodes[NA:], h0_tail, pos[NA:], noise[NA:],
      Wt2, A, Bv, C, Wc, Wn, bn)

    return jnp.sum(out_a) + jnp.sum(out_b)


# speedup vs baseline: 1.0690x; 1.0690x over previous
import functools

import jax
import jax.numpy as jnp
from jax import lax
from jax.experimental import pallas as pl
from jax.experimental.pallas import tpu as pltpu
from jax.experimental.pallas import tpu_sc as plsc

SMIN = 0.01
SMAX = 50.0
NUM_LAYERS = 2
MB = 64

SC_NC = 2
SC_NS = 16
SC_NW = SC_NC * SC_NS


def _sc_gather(table, idx, n_rows, D):
    b_per_w = n_rows // SC_NW
    mesh = plsc.VectorSubcoreMesh(core_axis_name="c", subcore_axis_name="s")

    @functools.partial(
        pl.kernel, mesh=mesh,
        out_type=jax.ShapeDtypeStruct((n_rows, D), jnp.float32),
        scratch_types=[
            pltpu.VMEM((b_per_w,), jnp.int32),
            pltpu.VMEM((b_per_w, D), jnp.float32),
            pltpu.SemaphoreType.DMA,
        ],
    )
    def k(table_hbm, idx_hbm, out_hbm, idx_v, rows_v, sem):
        wid = lax.axis_index("s") * SC_NC + lax.axis_index("c")
        base = wid * b_per_w
        pltpu.sync_copy(idx_hbm.at[pl.ds(base, b_per_w)], idx_v)
        pltpu.async_copy(table_hbm.at[idx_v], rows_v, sem).wait()
        pltpu.sync_copy(rows_v, out_hbm.at[pl.ds(base, b_per_w)])

    return k(table, idx)


def _step(t_ref, h0_ref, pos_ref, noise_ref, Wt_ref, A_ref, B_ref,
          C_ref, Wc_ref, Wn_ref, bn_ref, out_ref, *, mb, n, D, n_total):
    NB = mb * n

    t_nodes = t_ref[...]
    std = SMIN * (SMAX / SMIN) ** t_nodes

    noise = noise_ref[...]
    noise3 = noise.reshape(mb, n, 3)
    noise_c = (noise3 - jnp.mean(noise3, axis=1, keepdims=True)).reshape(NB, 3)

    x = pos_ref[...] + noise_c * std

    h = h0_ref[...] + t_nodes * Wt_ref[0][None, :]

    score = jnp.zeros((NB, 3), dtype=jnp.float32)
    for l in range(NUM_LAYERS):
        a = h * A_ref[l][None, :]
        b = h * B_ref[l][None, :]
        Cl = C_ref[l][None, :]
        Wcl = Wc_ref[l].reshape(D, 1)
        a3 = a.reshape(mb, n, D)
        x3 = x.reshape(mb, n, 3)
        agg_m = jnp.zeros((NB, D), dtype=jnp.float32)
        agg_x = jnp.zeros((NB, 3), dtype=jnp.float32)
        for i in range(n):
            a_i = jnp.broadcast_to(a3[:, i:i + 1, :], (mb, n, D))
            x_i = jnp.broadcast_to(x3[:, i:i + 1, :], (mb, n, 3))
            rel = (x_i - x3).reshape(NB, 3)
            d2 = jnp.sum(rel * rel, axis=1, keepdims=True)
            m = jax.nn.silu(a_i.reshape(NB, D) + b + d2 * Cl)
            agg_m = agg_m + m
            coef = jax.lax.dot_general(m, Wcl, (((1,), (0,)), ((), ())),
                                       preferred_element_type=jnp.float32)
            agg_x = agg_x + rel * coef
        agg_x = agg_x / n
        h = h + jax.nn.silu(
            jax.lax.dot_general(agg_m, Wn_ref[l], (((1,), (0,)), ((), ())),
                                preferred_element_type=jnp.float32)
            + bn_ref[l][None, :])
        x = x + agg_x
        score = score + agg_x

    score = score / std
    score3 = score.reshape(mb, n, 3)
    score = (score3 - jnp.mean(score3, axis=1, keepdims=True)).reshape(NB, 3)
    r = score * std + noise_c
    partial = jnp.sum(r * r, axis=(0, 1), keepdims=True) / n_total

    @pl.when(pl.program_id(0) == 0)
    def _init():
        out_ref[...] = jnp.zeros((1, 1), jnp.float32)

    out_ref[...] += partial


def kernel(pos, atomic_numbers, mask, atom_emb, W_t, A, Bv, C, Wc, Wn, bn):
    B = mask.shape[0]
    N = pos.shape[0]
    n = N // B
    D = atom_emb.shape[1]

    kt = jax.random.fold_in(jax.random.key(0), 1)
    kn = jax.random.fold_in(jax.random.key(0), 2)
    t = jax.random.uniform(kt, (B,), minval=1e-3, maxval=1.0, dtype=jnp.float32)
    noise = jax.random.normal(kn, (N, 3), dtype=jnp.float32)

    t_nodes = jnp.repeat(t, n).reshape(N, 1)
    Wt2 = W_t.reshape(1, D)

    h0 = _sc_gather(atom_emb, atomic_numbers, N, D)

    mb = MB
    grid = B // mb
    NB = mb * n
    full = lambda g: (0, 0)
    out = pl.pallas_call(
        functools.partial(_step, mb=mb, n=n, D=D, n_total=N),
        grid=(grid,),
        in_specs=[
            pl.BlockSpec((NB, 1), lambda g: (g, 0)),
            pl.BlockSpec((NB, D), lambda g: (g, 0)),
            pl.BlockSpec((NB, 3), lambda g: (g, 0)),
            pl.BlockSpec((NB, 3), lambda g: (g, 0)),
            pl.BlockSpec((1, D), full),
            pl.BlockSpec((NUM_LAYERS, D), full),
            pl.BlockSpec((NUM_LAYERS, D), full),
            pl.BlockSpec((NUM_LAYERS, D), full),
            pl.BlockSpec((NUM_LAYERS, D), full),
            pl.BlockSpec((NUM_LAYERS, D, D), lambda g: (0, 0, 0)),
            pl.BlockSpec((NUM_LAYERS, D), full),
        ],
        out_specs=pl.BlockSpec((1, 1), full),
        out_shape=jax.ShapeDtypeStruct((1, 1), jnp.float32),
    )(t_nodes, h0, pos, noise, Wt2, A, Bv, C, Wc, Wn, bn)
    return out[0, 0]
